# Initial kernel scaffold; baseline (speedup 1.0000x reference)
import jax
import jax.numpy as jnp
from jax.experimental import pallas as pl


def kernel(x, packed_weight, norms, signs1, signs2, centroids, bias):
    raise NotImplementedError("write your pallas kernel here")


# speedup vs baseline: 1.5514x; 1.5514x over previous
import numpy as np
import jax
import jax.numpy as jnp
from jax.experimental import pallas as pl
from jax.experimental.pallas import tpu as pltpu

_N_CODES = 8

_DQ_ROWS = 8192
_BM = 512
_BN = 1024


def _butterfly_matrix(g: int) -> np.ndarray:
    x = np.eye(g, dtype=np.float64)
    for _ in range(int(np.log2(g))):
        x = x.reshape(x.shape[:-1] + (2, g // 2))
        a, b = x[..., 0, :], x[..., 1, :]
        x = np.concatenate([a + b, a - b], axis=-1)
    return x / np.sqrt(g)


def _dq_body(idx_ref, nrm_ref, cent_ref, s_ref, w_ref):
    idx = idx_ref[...]
    v = jnp.full(idx.shape, cent_ref[0], dtype=jnp.float32)
    for c in range(1, _N_CODES):
        v = jnp.where(idx == c, cent_ref[c], v)
    v = v * nrm_ref[...]
    w = jax.lax.dot(v.astype(jnp.bfloat16), s_ref[...],
                    preferred_element_type=jnp.float32)
    w_ref[...] = w.astype(jnp.bfloat16)


def _mm_body(x_ref, w_ref, b_ref, o_ref):
    xb = x_ref[...].astype(jnp.bfloat16)
    acc = jax.lax.dot_general(xb, w_ref[...], (((1,), (1,)), ((), ())),
                              preferred_element_type=jnp.float32)
    o_ref[...] = acc + b_ref[...]


def kernel(x, packed_weight, norms, signs1, signs2, centroids, bias):
    b, s, in_dim = x.shape
    out_dim, n_groups = norms.shape
    g = packed_weight.shape[-1]
    n_rows = out_dim * n_groups

    bmat = jnp.asarray(_butterfly_matrix(g), dtype=jnp.float32)
    smat = (signs2[:, None] * bmat * signs1[None, :]).astype(jnp.bfloat16)
    nrm2 = norms.reshape(n_rows, 1)

    dq_rows = min(_DQ_ROWS, n_rows)
    w_big = pl.pallas_call(
        _dq_body,
        grid=(n_rows // dq_rows,),
        in_specs=[
            pl.BlockSpec((dq_rows, g), lambda i: (i, 0)),
            pl.BlockSpec((dq_rows, 1), lambda i: (i, 0)),
            pl.BlockSpec(memory_space=pltpu.SMEM),
            pl.BlockSpec((g, g), lambda i: (0, 0)),
        ],
        out_specs=pl.BlockSpec((dq_rows, g), lambda i: (i, 0)),
        out_shape=jax.ShapeDtypeStruct((n_rows, g), jnp.bfloat16),
        compiler_params=pltpu.CompilerParams(
            dimension_semantics=("parallel",)),
    )(packed_weight, nrm2, centroids, smat)

    w = w_big.reshape(out_dim, in_dim)
    x2 = x.reshape(b * s, in_dim)
    bias2 = bias.reshape(1, out_dim)

    bm = min(_BM, b * s)
    out2 = pl.pallas_call(
        _mm_body,
        grid=(out_dim // _BN, (b * s) // bm),
        in_specs=[
            pl.BlockSpec((bm, in_dim), lambda n, m: (m, 0)),
            pl.BlockSpec((_BN, in_dim), lambda n, m: (n, 0)),
            pl.BlockSpec((1, _BN), lambda n, m: (0, n)),
        ],
        out_specs=pl.BlockSpec((bm, _BN), lambda n, m: (m, n)),
        out_shape=jax.ShapeDtypeStruct((b * s, out_dim), jnp.float32),
        compiler_params=pltpu.CompilerParams(
            dimension_semantics=("parallel", "parallel"),
            vmem_limit_bytes=56 * 1024 * 1024),
    )(x2, w, bias2)

    return out2.reshape(b, s, out_dim)


# speedup vs baseline: 1.6050x; 1.0345x over previous
import numpy as np
import jax
import jax.numpy as jnp
from jax.experimental import pallas as pl
from jax.experimental.pallas import tpu as pltpu

_N_CODES = 8

_DQ_ROWS = 8192
_BM = 256


def _butterfly_matrix(g: int) -> np.ndarray:
    x = np.eye(g, dtype=np.float64)
    for _ in range(int(np.log2(g))):
        x = x.reshape(x.shape[:-1] + (2, g // 2))
        a, b = x[..., 0, :], x[..., 1, :]
        x = np.concatenate([a + b, a - b], axis=-1)
    return x / np.sqrt(g)


def _dq_body(idx_ref, nrm_ref, cent_ref, s_ref, w_ref):
    idx = idx_ref[...]
    v = jnp.full(idx.shape, cent_ref[0], dtype=jnp.float32)
    for c in range(1, _N_CODES):
        v = jnp.where(idx == c, cent_ref[c], v)
    v = v * nrm_ref[...]
    w = jax.lax.dot(v.astype(jnp.bfloat16), s_ref[...],
                    preferred_element_type=jnp.float32)
    w_ref[...] = w.astype(jnp.bfloat16)


def _mm_body(x_ref, w_ref, b_ref, o_ref):
    xb = x_ref[...].astype(jnp.bfloat16)
    acc = jax.lax.dot_general(xb, w_ref[...], (((1,), (1,)), ((), ())),
                              preferred_element_type=jnp.float32)
    o_ref[...] = acc + b_ref[...]


def kernel(x, packed_weight, norms, signs1, signs2, centroids, bias):
    b, s, in_dim = x.shape
    out_dim, n_groups = norms.shape
    g = packed_weight.shape[-1]
    n_rows = out_dim * n_groups

    bmat = jnp.asarray(_butterfly_matrix(g), dtype=jnp.float32)
    smat = (signs2[:, None] * bmat * signs1[None, :]).astype(jnp.bfloat16)
    nrm2 = norms.reshape(n_rows, 1)

    dq_rows = min(_DQ_ROWS, n_rows)
    w_big = pl.pallas_call(
        _dq_body,
        grid=(n_rows // dq_rows,),
        in_specs=[
            pl.BlockSpec((dq_rows, g), lambda i: (i, 0)),
            pl.BlockSpec((dq_rows, 1), lambda i: (i, 0)),
            pl.BlockSpec(memory_space=pltpu.SMEM),
            pl.BlockSpec((g, g), lambda i: (0, 0)),
        ],
        out_specs=pl.BlockSpec((dq_rows, g), lambda i: (i, 0)),
        out_shape=jax.ShapeDtypeStruct((n_rows, g), jnp.bfloat16),
        compiler_params=pltpu.CompilerParams(
            dimension_semantics=("parallel",)),
    )(packed_weight, nrm2, centroids, smat)

    w = w_big.reshape(out_dim, in_dim)
    x2 = x.reshape(b * s, in_dim)
    bias2 = bias.reshape(1, out_dim)

    bm = min(_BM, b * s)
    out2 = pl.pallas_call(
        _mm_body,
        grid=((b * s) // bm,),
        in_specs=[
            pl.BlockSpec((bm, in_dim), lambda m: (m, 0)),
            pl.BlockSpec((out_dim, in_dim), lambda m: (0, 0)),
            pl.BlockSpec((1, out_dim), lambda m: (0, 0)),
        ],
        out_specs=pl.BlockSpec((bm, out_dim), lambda m: (m, 0)),
        out_shape=jax.ShapeDtypeStruct((b * s, out_dim), jnp.float32),
        compiler_params=pltpu.CompilerParams(
            dimension_semantics=("parallel",),
            vmem_limit_bytes=57 * 1024 * 1024),
    )(x2, w, bias2)

    return out2.reshape(b, s, out_dim)
